# baseline (speedup 1.0000x reference)
import jax
import jax.numpy as jnp
from jax.experimental import pallas as pl
from jax.experimental.pallas import tpu as pltpu

BATCH = 16384
VOCAB = 1000
EMBED = 16
BLOCK_N = 2048


def _body(w_ref, x_ref, o_ref):
    wb = w_ref[...].astype(jnp.bfloat16)
    xb = x_ref[...].astype(jnp.bfloat16)
    o_ref[...] = jax.lax.dot_general(
        xb, wb,
        dimension_numbers=(((0,), (1,)), ((), ())),
        preferred_element_type=jnp.float32,
        precision=jax.lax.Precision.DEFAULT,
    )


def kernel(one_hot, weight):
    x_t = one_hot.T
    w_t = weight.T
    grid = (BATCH // BLOCK_N,)
    return pl.pallas_call(
        _body,
        grid=grid,
        in_specs=[
            pl.BlockSpec((EMBED, VOCAB), lambda i: (0, 0)),
            pl.BlockSpec((VOCAB, BLOCK_N), lambda i: (0, i)),
        ],
        out_specs=pl.BlockSpec((BLOCK_N, EMBED), lambda i: (i, 0)),
        out_shape=jax.ShapeDtypeStruct((BATCH, EMBED), jnp.float32),
        compiler_params=pltpu.CompilerParams(
            dimension_semantics=("arbitrary",),
        ),
    )(w_t, x_t)


# speedup vs baseline: 1.2963x; 1.2963x over previous
import jax
import jax.numpy as jnp
from jax.experimental import pallas as pl
from jax.experimental.pallas import tpu as pltpu

BATCH = 16384
VOCAB = 1000
EMBED = 16
BLOCK_N = 2048
HALF = BLOCK_N // 2


def _body(w_ref, x0_ref, x1_ref, o_ref):
    wb = w_ref[...].astype(jnp.bfloat16)
    for j, x_ref in enumerate((x0_ref, x1_ref)):
        xb = x_ref[...].astype(jnp.bfloat16)
        o_ref[:, pl.ds(j * HALF, HALF)] = jax.lax.dot_general(
            wb, xb,
            dimension_numbers=(((1,), (0,)), ((), ())),
            preferred_element_type=jnp.float32,
            precision=jax.lax.Precision.DEFAULT,
        )


def kernel(one_hot, weight):
    x_t = one_hot.T
    w_t = weight.T
    grid = (BATCH // BLOCK_N,)
    out_t = pl.pallas_call(
        _body,
        grid=grid,
        in_specs=[
            pl.BlockSpec((EMBED, VOCAB), lambda i: (0, 0)),
            pl.BlockSpec((VOCAB, HALF), lambda i: (0, 2 * i)),
            pl.BlockSpec((VOCAB, HALF), lambda i: (0, 2 * i + 1)),
        ],
        out_specs=pl.BlockSpec((EMBED, BLOCK_N), lambda i: (0, i)),
        out_shape=jax.ShapeDtypeStruct((EMBED, BATCH), jnp.float32),
        compiler_params=pltpu.CompilerParams(
            dimension_semantics=("arbitrary",),
        ),
    )(w_t, x_t, x_t)
    return out_t.T
